# Initial kernel scaffold; baseline (speedup 1.0000x reference)
import functools

import jax
import jax.numpy as jnp
from jax import lax
from jax.experimental import pallas as pl
from jax.experimental.pallas import tpu as pltpu
from jax.experimental.pallas import tpu_sc as plsc

_N = 10000
_E = 320000
_D = 128
_SOP = 512
_G = 16

_NB = 10
_BN = _N // _NB
_EB = _E // _NB

_NT = 32
_TPS = _E // _NT
_K = 80
_NCH = _TPS // _K
_NPAD = 10240
_RPT = _N // 16


def _leaky(z):
    return jnp.where(z >= 0.0, z, 0.2 * z)


def _prep1_body(x_ref, w1_ref, asv_ref, adv_ref, we1_ref, ae1_ref, we2_ref,
                ae2_ref, ea_ref, h_ref, as_ref, ad_ref, c1_ref, c2_ref,
                mea_ref):
    i = pl.program_id(0)
    h = jnp.dot(x_ref[...], w1_ref[...], precision=lax.Precision.HIGHEST)
    h_ref[...] = h
    as_ref[...] = jnp.sum(h * asv_ref[...], axis=1, keepdims=True)
    ad_ref[...] = jnp.sum(h * adv_ref[...], axis=1, keepdims=True)

    @pl.when(i == 0)
    def _():
        c1_ref[0, 0] = jnp.sum(we1_ref[...] * ae1_ref[...])
        c2_ref[0, 0] = jnp.sum(we2_ref[...] * ae2_ref[...])
        mea_ref[0, 0] = 0.0

    mea_ref[0, 0] += jnp.sum(ea_ref[...])

    @pl.when(i == _NB - 1)
    def _():
        mea_ref[0, 0] = mea_ref[0, 0] / _E


_prep1 = pl.pallas_call(
    _prep1_body,
    grid=(_NB,),
    in_specs=[
        pl.BlockSpec((_BN, _D), lambda i: (i, 0)),
        pl.BlockSpec((_D, _D), lambda i: (0, 0)),
        pl.BlockSpec((1, _D), lambda i: (0, 0)),
        pl.BlockSpec((1, _D), lambda i: (0, 0)),
        pl.BlockSpec((1, _D), lambda i: (0, 0)),
        pl.BlockSpec((1, _D), lambda i: (0, 0)),
        pl.BlockSpec((1, _D), lambda i: (0, 0)),
        pl.BlockSpec((1, _D), lambda i: (0, 0)),
        pl.BlockSpec((_EB, 1), lambda i: (i, 0)),
    ],
    out_specs=[
        pl.BlockSpec((_BN, _D), lambda i: (i, 0)),
        pl.BlockSpec((_BN, 1), lambda i: (i, 0)),
        pl.BlockSpec((_BN, 1), lambda i: (i, 0)),
        pl.BlockSpec((1, 1), lambda i: (0, 0)),
        pl.BlockSpec((1, 1), lambda i: (0, 0)),
        pl.BlockSpec((1, 1), lambda i: (0, 0)),
    ],
    out_shape=[
        jax.ShapeDtypeStruct((_N, _D), jnp.float32),
        jax.ShapeDtypeStruct((_N, 1), jnp.float32),
        jax.ShapeDtypeStruct((_N, 1), jnp.float32),
        jax.ShapeDtypeStruct((1, 1), jnp.float32),
        jax.ShapeDtypeStruct((1, 1), jnp.float32),
        jax.ShapeDtypeStruct((1, 1), jnp.float32),
    ],
)


def _mid_body(num_ref, den_ref, h_ref, as1_ref, ad1_ref, c1_ref, mea_ref,
              b1_ref, w2_ref, asv2_ref, adv2_ref, h2_ref, as2_ref, ad2_ref):
    wself = jnp.exp(_leaky(as1_ref[...] + ad1_ref[...]
                           + c1_ref[0, 0] * mea_ref[0, 0]))
    h = h_ref[...]
    num = num_ref[0] + num_ref[1] + wself * h
    den = den_ref[0] + den_ref[1] + wself + 1e-16
    h1a = jnp.maximum(num / den + b1_ref[...], 0.0)
    h2 = jnp.dot(h1a, w2_ref[...], precision=lax.Precision.HIGHEST)
    h2_ref[...] = h2
    as2_ref[...] = jnp.sum(h2 * asv2_ref[...], axis=1, keepdims=True)
    ad2_ref[...] = jnp.sum(h2 * adv2_ref[...], axis=1, keepdims=True)


_mid = pl.pallas_call(
    _mid_body,
    grid=(_NB,),
    in_specs=[
        pl.BlockSpec((2, _BN, _D), lambda i: (0, i, 0)),
        pl.BlockSpec((2, _BN, 1), lambda i: (0, i, 0)),
        pl.BlockSpec((_BN, _D), lambda i: (i, 0)),
        pl.BlockSpec((_BN, 1), lambda i: (i, 0)),
        pl.BlockSpec((_BN, 1), lambda i: (i, 0)),
        pl.BlockSpec((1, 1), lambda i: (0, 0)),
        pl.BlockSpec((1, 1), lambda i: (0, 0)),
        pl.BlockSpec((1, _D), lambda i: (0, 0)),
        pl.BlockSpec((_D, _D), lambda i: (0, 0)),
        pl.BlockSpec((1, _D), lambda i: (0, 0)),
        pl.BlockSpec((1, _D), lambda i: (0, 0)),
    ],
    out_specs=[
        pl.BlockSpec((_BN, _D), lambda i: (i, 0)),
        pl.BlockSpec((_BN, 1), lambda i: (i, 0)),
        pl.BlockSpec((_BN, 1), lambda i: (i, 0)),
    ],
    out_shape=[
        jax.ShapeDtypeStruct((_N, _D), jnp.float32),
        jax.ShapeDtypeStruct((_N, 1), jnp.float32),
        jax.ShapeDtypeStruct((_N, 1), jnp.float32),
    ],
)


def _final_body(num_ref, den_ref, h_ref, as2_ref, ad2_ref, c2_ref, mea_ref,
                b2_ref, batch_ref, s1w_ref, s1b_ref, s2w_ref, s2b_ref,
                s3w_ref, s3b_ref, gp_ref, gl_ref, acc_v, cnt_v):
    i = pl.program_id(0)
    wself = jnp.exp(_leaky(as2_ref[...] + ad2_ref[...]
                           + c2_ref[0, 0] * mea_ref[0, 0]))
    h = h_ref[...]
    num = num_ref[0] + num_ref[1] + wself * h
    den = den_ref[0] + den_ref[1] + wself + 1e-16
    h2a = jnp.maximum(num / den + b2_ref[...], 0.0)
    nrm = jnp.sqrt(jnp.sum(h2a * h2a, axis=1, keepdims=True))
    nz = h2a / jnp.maximum(nrm, 1e-12)

    g = jnp.maximum(jnp.dot(nz, s1w_ref[...],
                            precision=lax.Precision.HIGHEST) + s1b_ref[...], 0.0)
    g = jnp.maximum(jnp.dot(g, s2w_ref[...],
                            precision=lax.Precision.HIGHEST) + s2b_ref[...], 0.0)
    gp_ref[...] = jnp.maximum(jnp.dot(g, s3w_ref[...],
                                      precision=lax.Precision.HIGHEST)
                              + s3b_ref[...], 0.0)

    b = batch_ref[0, 0, :]
    oh = (b[:, None] == lax.broadcasted_iota(jnp.int32, (_BN, _G), 1)
          ).astype(jnp.float32)
    part = lax.dot_general(oh, nz, (((0,), (0,)), ((), ())),
                           precision=lax.Precision.HIGHEST)
    cnt = jnp.sum(oh, axis=0)[:, None]

    @pl.when(i == 0)
    def _():
        acc_v[...] = jnp.zeros((_G, _D), jnp.float32)
        cnt_v[...] = jnp.zeros((_G, _D), jnp.float32)

    acc_v[...] += part
    cnt_v[...] += jnp.broadcast_to(cnt, (_G, _D))

    @pl.when(i == _NB - 1)
    def _():
        gl_ref[...] = acc_v[...] / jnp.maximum(cnt_v[...], 1.0)


_final = pl.pallas_call(
    _final_body,
    grid=(_NB,),
    in_specs=[
        pl.BlockSpec((2, _BN, _D), lambda i: (0, i, 0)),
        pl.BlockSpec((2, _BN, 1), lambda i: (0, i, 0)),
        pl.BlockSpec((_BN, _D), lambda i: (i, 0)),
        pl.BlockSpec((_BN, 1), lambda i: (i, 0)),
        pl.BlockSpec((_BN, 1), lambda i: (i, 0)),
        pl.BlockSpec((1, 1), lambda i: (0, 0)),
        pl.BlockSpec((1, 1), lambda i: (0, 0)),
        pl.BlockSpec((1, _D), lambda i: (0, 0)),
        pl.BlockSpec((1, 1, _BN), lambda i: (i, 0, 0)),
        pl.BlockSpec((_D, _SOP), lambda i: (0, 0)),
        pl.BlockSpec((1, _SOP), lambda i: (0, 0)),
        pl.BlockSpec((_SOP, _SOP), lambda i: (0, 0)),
        pl.BlockSpec((1, _SOP), lambda i: (0, 0)),
        pl.BlockSpec((_SOP, _SOP), lambda i: (0, 0)),
        pl.BlockSpec((1, _SOP), lambda i: (0, 0)),
    ],
    out_specs=[
        pl.BlockSpec((_BN, _SOP), lambda i: (i, 0)),
        pl.BlockSpec((_G, _D), lambda i: (0, 0)),
    ],
    out_shape=[
        jax.ShapeDtypeStruct((_N, _SOP), jnp.float32),
        jax.ShapeDtypeStruct((_G, _D), jnp.float32),
    ],
    scratch_shapes=[
        pltpu.VMEM((_G, _D), jnp.float32),
        pltpu.VMEM((_G, _D), jnp.float32),
    ],
)


def _edge_body(src_hbm, dst_hbm, ea_hbm, zrow_hbm, zden_hbm, c_hbm, asrc_hbm,
               adst_hbm, h_hbm, num_out, den_out, asrc_v, adst_v, cvec_v,
               src_v, dst_v, ea_v, w_v, rows_v, num_sh, den_sh, sem):
    cid = lax.axis_index("c")
    sid = lax.axis_index("s")
    wid = cid * 16 + sid

    pltpu.sync_copy(asrc_hbm, asrc_v)
    pltpu.sync_copy(adst_hbm, adst_v)
    pltpu.sync_copy(c_hbm, cvec_v)

    for r in range(_RPT // 125):
        pltpu.sync_copy(zrow_hbm,
                        num_sh.at[pl.ds(sid * _RPT + r * 125, 125), :])
    pltpu.sync_copy(zden_hbm, den_sh.at[pl.ds(sid * 640, 640)])
    plsc.subcore_barrier()

    cv = cvec_v[...]

    def chunk(g, carry):
        base = wid * _TPS + g * _K
        pltpu.sync_copy(src_hbm.at[pl.ds(base, _K)], src_v)
        pltpu.sync_copy(dst_hbm.at[pl.ds(base, _K)], dst_v)
        pltpu.sync_copy(ea_hbm.at[pl.ds(base, _K)], ea_v)
        pltpu.async_copy(h_hbm.at[src_v], rows_v, sem).wait()
        for gg in range(_K // 16):
            s16 = src_v[pl.ds(gg * 16, 16)]
            d16 = dst_v[pl.ds(gg * 16, 16)]
            e16 = ea_v[pl.ds(gg * 16, 16)]
            z = (plsc.load_gather(asrc_v, [s16])
                 + plsc.load_gather(adst_v, [d16]) + cv * e16)
            z = jnp.where(z >= 0.0, z, 0.2 * z)
            w_v[pl.ds(gg * 16, 16)] = jnp.exp(z)
        for e in range(_K):
            wb = jnp.full((16,), w_v[e], jnp.float32)
            for j in range(_D // 16):
                rows_v[e, pl.ds(j * 16, 16)] = rows_v[e, pl.ds(j * 16, 16)] * wb
        pltpu.sync_copy(rows_v, num_sh.at[dst_v], add=True)
        pltpu.sync_copy(w_v, den_sh.at[dst_v], add=True)
        return carry

    lax.fori_loop(0, _NCH, chunk, 0)
    plsc.subcore_barrier()

    for r in range(_RPT // 125):
        pltpu.sync_copy(num_sh.at[pl.ds(sid * _RPT + r * 125, 125), :],
                        num_out.at[cid, pl.ds(sid * _RPT + r * 125, 125), :])
    pltpu.sync_copy(den_sh.at[pl.ds(sid * 640, 640)],
                    den_out.at[cid, pl.ds(sid * 640, 640)])


_edge_pass = pl.kernel(
    _edge_body,
    out_type=(jax.ShapeDtypeStruct((2, _N, _D), jnp.float32),
              jax.ShapeDtypeStruct((2, _NPAD), jnp.float32)),
    mesh=plsc.VectorSubcoreMesh(core_axis_name="c", subcore_axis_name="s"),
    scratch_types=[
        pltpu.VMEM((_N,), jnp.float32),
        pltpu.VMEM((_N,), jnp.float32),
        pltpu.VMEM((16,), jnp.float32),
        pltpu.VMEM((_K,), jnp.int32),
        pltpu.VMEM((_K,), jnp.int32),
        pltpu.VMEM((_K,), jnp.float32),
        pltpu.VMEM((_K,), jnp.float32),
        pltpu.VMEM((_K, _D), jnp.float32),
        pltpu.VMEM_SHARED((_N, _D), jnp.float32),
        pltpu.VMEM_SHARED((_NPAD,), jnp.float32),
        pltpu.SemaphoreType.DMA,
    ],
)


def kernel(x, edge_index, edge_attr, batch, W1, att_src1, att_dst1, W_edge1,
           att_edge1, bias1, W2, att_src2, att_dst2, W_edge2, att_edge2,
           bias2, S1_w, S1_b, S2_w, S2_b, S3_w, S3_b):
    src = edge_index[0]
    dst = edge_index[1]
    ea = edge_attr[:, 0]

    h1, as1, ad1, c1, c2, mea = _prep1(
        x, W1, att_src1.reshape(1, _D), att_dst1.reshape(1, _D),
        W_edge1.reshape(1, _D), att_edge1.reshape(1, _D),
        W_edge2.reshape(1, _D), att_edge2.reshape(1, _D), edge_attr)

    zrow = jnp.zeros((125, _D), jnp.float32)
    zden = jnp.zeros((640,), jnp.float32)
    c1v = jnp.broadcast_to(c1.reshape(()), (16,))
    c2v = jnp.broadcast_to(c2.reshape(()), (16,))

    num1, den1 = _edge_pass(src, dst, ea, zrow, zden, c1v,
                            as1.reshape(_N), ad1.reshape(_N), h1)

    h2, as2, ad2 = _mid(num1, den1[:, :_N, None], h1, as1, ad1, c1, mea,
                        bias1.reshape(1, _D), W2, att_src2.reshape(1, _D),
                        att_dst2.reshape(1, _D))

    num2, den2 = _edge_pass(src, dst, ea, zrow, zden, c2v,
                            as2.reshape(_N), ad2.reshape(_N), h2)

    gp, gl = _final(num2, den2[:, :_N, None], h2, as2, ad2, c2, mea,
                    bias2.reshape(1, _D), batch.reshape(_NB, 1, _BN),
                    S1_w, S1_b.reshape(1, _SOP), S2_w, S2_b.reshape(1, _SOP),
                    S3_w, S3_b.reshape(1, _SOP))
    return (gp, gl)


# speedup vs baseline: 18.7885x; 18.7885x over previous
import functools

import jax
import jax.numpy as jnp
from jax import lax
from jax.experimental import pallas as pl
from jax.experimental.pallas import tpu as pltpu
from jax.experimental.pallas import tpu_sc as plsc

_N = 10000
_E = 320000
_D = 128
_SOP = 512
_G = 16

_NB = 10
_BN = _N // _NB
_EB = _E // _NB

_NT = 32
_TPS = _E // _NT
_K = 80
_NCH = _TPS // _K
_NPAD = 10240
_RPT = _NPAD // 16


def _leaky(z):
    return jnp.where(z >= 0.0, z, 0.2 * z)


def _prep1_body(x_ref, w1_ref, asv_ref, adv_ref, we1_ref, ae1_ref, we2_ref,
                ae2_ref, ea_ref, h_ref, as_ref, ad_ref, c1_ref, c2_ref,
                mea_ref):
    i = pl.program_id(0)
    h = jnp.dot(x_ref[...], w1_ref[...], precision=lax.Precision.HIGHEST)
    h_ref[...] = h
    as_ref[...] = jnp.sum(h * asv_ref[...], axis=1, keepdims=True)
    ad_ref[...] = jnp.sum(h * adv_ref[...], axis=1, keepdims=True)

    @pl.when(i == 0)
    def _():
        c1_ref[0, 0] = jnp.sum(we1_ref[...] * ae1_ref[...])
        c2_ref[0, 0] = jnp.sum(we2_ref[...] * ae2_ref[...])
        mea_ref[0, 0] = 0.0

    mea_ref[0, 0] += jnp.sum(ea_ref[...])

    @pl.when(i == _NB - 1)
    def _():
        mea_ref[0, 0] = mea_ref[0, 0] / _E


_prep1 = pl.pallas_call(
    _prep1_body,
    grid=(_NB,),
    in_specs=[
        pl.BlockSpec((_BN, _D), lambda i: (i, 0)),
        pl.BlockSpec((_D, _D), lambda i: (0, 0)),
        pl.BlockSpec((1, _D), lambda i: (0, 0)),
        pl.BlockSpec((1, _D), lambda i: (0, 0)),
        pl.BlockSpec((1, _D), lambda i: (0, 0)),
        pl.BlockSpec((1, _D), lambda i: (0, 0)),
        pl.BlockSpec((1, _D), lambda i: (0, 0)),
        pl.BlockSpec((1, _D), lambda i: (0, 0)),
        pl.BlockSpec((_EB, 1), lambda i: (i, 0)),
    ],
    out_specs=[
        pl.BlockSpec((_BN, _D), lambda i: (i, 0)),
        pl.BlockSpec((_BN, 1), lambda i: (i, 0)),
        pl.BlockSpec((_BN, 1), lambda i: (i, 0)),
        pl.BlockSpec((1, 1), lambda i: (0, 0), memory_space=pltpu.SMEM),
        pl.BlockSpec((1, 1), lambda i: (0, 0), memory_space=pltpu.SMEM),
        pl.BlockSpec((1, 1), lambda i: (0, 0), memory_space=pltpu.SMEM),
    ],
    out_shape=[
        jax.ShapeDtypeStruct((_N, _D), jnp.float32),
        jax.ShapeDtypeStruct((_N, 1), jnp.float32),
        jax.ShapeDtypeStruct((_N, 1), jnp.float32),
        jax.ShapeDtypeStruct((1, 1), jnp.float32),
        jax.ShapeDtypeStruct((1, 1), jnp.float32),
        jax.ShapeDtypeStruct((1, 1), jnp.float32),
    ],
)


def _mid_body(num_ref, den_ref, h_ref, as1_ref, ad1_ref, c1_ref, mea_ref,
              b1_ref, w2_ref, asv2_ref, adv2_ref, h2_ref, as2_ref, ad2_ref):
    wself = jnp.exp(_leaky(as1_ref[...] + ad1_ref[...]
                           + c1_ref[0, 0] * mea_ref[0, 0]))
    h = h_ref[...]
    num = num_ref[0] + num_ref[1] + wself * h
    den = den_ref[0] + den_ref[1] + wself + 1e-16
    h1a = jnp.maximum(num / den + b1_ref[...], 0.0)
    h2 = jnp.dot(h1a, w2_ref[...], precision=lax.Precision.HIGHEST)
    h2_ref[...] = h2
    as2_ref[...] = jnp.sum(h2 * asv2_ref[...], axis=1, keepdims=True)
    ad2_ref[...] = jnp.sum(h2 * adv2_ref[...], axis=1, keepdims=True)


_mid = pl.pallas_call(
    _mid_body,
    grid=(_NB,),
    in_specs=[
        pl.BlockSpec((2, _BN, _D), lambda i: (0, i, 0)),
        pl.BlockSpec((2, _BN, 1), lambda i: (0, i, 0)),
        pl.BlockSpec((_BN, _D), lambda i: (i, 0)),
        pl.BlockSpec((_BN, 1), lambda i: (i, 0)),
        pl.BlockSpec((_BN, 1), lambda i: (i, 0)),
        pl.BlockSpec((1, 1), lambda i: (0, 0), memory_space=pltpu.SMEM),
        pl.BlockSpec((1, 1), lambda i: (0, 0), memory_space=pltpu.SMEM),
        pl.BlockSpec((1, _D), lambda i: (0, 0)),
        pl.BlockSpec((_D, _D), lambda i: (0, 0)),
        pl.BlockSpec((1, _D), lambda i: (0, 0)),
        pl.BlockSpec((1, _D), lambda i: (0, 0)),
    ],
    out_specs=[
        pl.BlockSpec((_BN, _D), lambda i: (i, 0)),
        pl.BlockSpec((_BN, 1), lambda i: (i, 0)),
        pl.BlockSpec((_BN, 1), lambda i: (i, 0)),
    ],
    out_shape=[
        jax.ShapeDtypeStruct((_N, _D), jnp.float32),
        jax.ShapeDtypeStruct((_N, 1), jnp.float32),
        jax.ShapeDtypeStruct((_N, 1), jnp.float32),
    ],
)


def _final_body(num_ref, den_ref, h_ref, as2_ref, ad2_ref, c2_ref, mea_ref,
                b2_ref, batch_ref, s1w_ref, s1b_ref, s2w_ref, s2b_ref,
                s3w_ref, s3b_ref, gp_ref, gl_ref, acc_v, cnt_v):
    i = pl.program_id(0)
    wself = jnp.exp(_leaky(as2_ref[...] + ad2_ref[...]
                           + c2_ref[0, 0] * mea_ref[0, 0]))
    h = h_ref[...]
    num = num_ref[0] + num_ref[1] + wself * h
    den = den_ref[0] + den_ref[1] + wself + 1e-16
    h2a = jnp.maximum(num / den + b2_ref[...], 0.0)
    nrm = jnp.sqrt(jnp.sum(h2a * h2a, axis=1, keepdims=True))
    nz = h2a / jnp.maximum(nrm, 1e-12)

    g = jnp.maximum(jnp.dot(nz, s1w_ref[...],
                            precision=lax.Precision.HIGHEST) + s1b_ref[...], 0.0)
    g = jnp.maximum(jnp.dot(g, s2w_ref[...],
                            precision=lax.Precision.HIGHEST) + s2b_ref[...], 0.0)
    gp_ref[...] = jnp.maximum(jnp.dot(g, s3w_ref[...],
                                      precision=lax.Precision.HIGHEST)
                              + s3b_ref[...], 0.0)

    b = batch_ref[0, 0, :]
    oh = (b[:, None] == lax.broadcasted_iota(jnp.int32, (_BN, _G), 1)
          ).astype(jnp.float32)
    part = lax.dot_general(oh, nz, (((0,), (0,)), ((), ())),
                           precision=lax.Precision.HIGHEST)
    cnt = jnp.sum(oh, axis=0)[:, None]

    @pl.when(i == 0)
    def _():
        acc_v[...] = jnp.zeros((_G, _D), jnp.float32)
        cnt_v[...] = jnp.zeros((_G, _D), jnp.float32)

    acc_v[...] += part
    cnt_v[...] += jnp.broadcast_to(cnt, (_G, _D))

    @pl.when(i == _NB - 1)
    def _():
        gl_ref[...] = acc_v[...] / jnp.maximum(cnt_v[...], 1.0)


_final = pl.pallas_call(
    _final_body,
    grid=(_NB,),
    in_specs=[
        pl.BlockSpec((2, _BN, _D), lambda i: (0, i, 0)),
        pl.BlockSpec((2, _BN, 1), lambda i: (0, i, 0)),
        pl.BlockSpec((_BN, _D), lambda i: (i, 0)),
        pl.BlockSpec((_BN, 1), lambda i: (i, 0)),
        pl.BlockSpec((_BN, 1), lambda i: (i, 0)),
        pl.BlockSpec((1, 1), lambda i: (0, 0), memory_space=pltpu.SMEM),
        pl.BlockSpec((1, 1), lambda i: (0, 0), memory_space=pltpu.SMEM),
        pl.BlockSpec((1, _D), lambda i: (0, 0)),
        pl.BlockSpec((1, 1, _BN), lambda i: (i, 0, 0)),
        pl.BlockSpec((_D, _SOP), lambda i: (0, 0)),
        pl.BlockSpec((1, _SOP), lambda i: (0, 0)),
        pl.BlockSpec((_SOP, _SOP), lambda i: (0, 0)),
        pl.BlockSpec((1, _SOP), lambda i: (0, 0)),
        pl.BlockSpec((_SOP, _SOP), lambda i: (0, 0)),
        pl.BlockSpec((1, _SOP), lambda i: (0, 0)),
    ],
    out_specs=[
        pl.BlockSpec((_BN, _SOP), lambda i: (i, 0)),
        pl.BlockSpec((_G, _D), lambda i: (0, 0)),
    ],
    out_shape=[
        jax.ShapeDtypeStruct((_N, _SOP), jnp.float32),
        jax.ShapeDtypeStruct((_G, _D), jnp.float32),
    ],
    scratch_shapes=[
        pltpu.VMEM((_G, _D), jnp.float32),
        pltpu.VMEM((_G, _D), jnp.float32),
    ],
)


def _edge_body(src_hbm, dst_hbm, ea_hbm, zrow_hbm, zden_hbm, c_hbm, asrc_hbm,
               adst_hbm, h_hbm, num_out, den_out, asrc_v, adst_v, cvec_v,
               src_v, dst_v, ea_v, w_v, rows_v, num_sh, den_sh, sem):
    cid = lax.axis_index("c")
    sid = lax.axis_index("s")
    wid = cid * 16 + sid

    pltpu.sync_copy(asrc_hbm, asrc_v)
    pltpu.sync_copy(adst_hbm, adst_v)
    pltpu.sync_copy(c_hbm, cvec_v)

    for r in range(_RPT // 128):
        pltpu.sync_copy(zrow_hbm,
                        num_sh.at[pl.ds(sid * _RPT + r * 128, 128), :])
    pltpu.sync_copy(zden_hbm, den_sh.at[pl.ds(sid * 640, 640)])
    plsc.subcore_barrier()

    cv = cvec_v[...]

    def chunk(g, carry):
        base = wid * _TPS + g * _K
        pltpu.sync_copy(src_hbm.at[pl.ds(base, _K)], src_v)
        pltpu.sync_copy(dst_hbm.at[pl.ds(base, _K)], dst_v)
        pltpu.sync_copy(ea_hbm.at[pl.ds(base, _K)], ea_v)
        pltpu.async_copy(h_hbm.at[src_v], rows_v, sem).wait()
        for gg in range(_K // 16):
            s16 = src_v[pl.ds(gg * 16, 16)]
            d16 = dst_v[pl.ds(gg * 16, 16)]
            e16 = ea_v[pl.ds(gg * 16, 16)]
            z = (plsc.load_gather(asrc_v, [s16])
                 + plsc.load_gather(adst_v, [d16]) + cv * e16)
            z = jnp.where(z >= 0.0, z, 0.2 * z)
            w_v[pl.ds(gg * 16, 16)] = jnp.exp(z)
        for gg in range(_K // 16):
            wvec = w_v[pl.ds(gg * 16, 16)]
            for l in range(16):
                e = gg * 16 + l
                wb = jnp.full((16,), wvec[l], jnp.float32)
                for j in range(_D // 16):
                    rows_v[e, pl.ds(j * 16, 16)] = (
                        rows_v[e, pl.ds(j * 16, 16)] * wb)
        pltpu.sync_copy(rows_v, num_sh.at[dst_v], add=True)
        pltpu.sync_copy(w_v, den_sh.at[dst_v], add=True)
        return carry

    lax.fori_loop(0, _NCH, chunk, 0)
    plsc.subcore_barrier()

    for r in range(_RPT // 128):
        pltpu.sync_copy(num_sh.at[pl.ds(sid * _RPT + r * 128, 128), :],
                        num_out.at[cid, pl.ds(sid * _RPT + r * 128, 128), :])
    pltpu.sync_copy(den_sh.at[pl.ds(sid * 640, 640)],
                    den_out.at[cid, pl.ds(sid * 640, 640)])


_edge_pass = pl.kernel(
    _edge_body,
    out_type=(jax.ShapeDtypeStruct((2, _NPAD, _D), jnp.float32),
              jax.ShapeDtypeStruct((2, _NPAD), jnp.float32)),
    mesh=plsc.VectorSubcoreMesh(core_axis_name="c", subcore_axis_name="s",
                                num_cores=2, num_subcores=16),
    compiler_params=pltpu.CompilerParams(needs_layout_passes=False),
    scratch_types=[
        pltpu.VMEM((_N,), jnp.float32),
        pltpu.VMEM((_N,), jnp.float32),
        pltpu.VMEM((16,), jnp.float32),
        pltpu.VMEM((_K,), jnp.int32),
        pltpu.VMEM((_K,), jnp.int32),
        pltpu.VMEM((_K,), jnp.float32),
        pltpu.VMEM((_K,), jnp.float32),
        pltpu.VMEM((_K, _D), jnp.float32),
        pltpu.VMEM_SHARED((_NPAD, _D), jnp.float32),
        pltpu.VMEM_SHARED((_NPAD,), jnp.float32),
        pltpu.SemaphoreType.DMA,
    ],
)


def kernel(x, edge_index, edge_attr, batch, W1, att_src1, att_dst1, W_edge1,
           att_edge1, bias1, W2, att_src2, att_dst2, W_edge2, att_edge2,
           bias2, S1_w, S1_b, S2_w, S2_b, S3_w, S3_b):
    src = edge_index[0]
    dst = edge_index[1]
    ea = edge_attr[:, 0]

    h1, as1, ad1, c1, c2, mea = _prep1(
        x, W1, att_src1.reshape(1, _D), att_dst1.reshape(1, _D),
        W_edge1.reshape(1, _D), att_edge1.reshape(1, _D),
        W_edge2.reshape(1, _D), att_edge2.reshape(1, _D), edge_attr)

    zrow = jnp.zeros((128, _D), jnp.float32)
    zden = jnp.zeros((640,), jnp.float32)
    c1v = jnp.broadcast_to(c1.reshape(()), (16,))
    c2v = jnp.broadcast_to(c2.reshape(()), (16,))

    num1, den1 = _edge_pass(src, dst, ea, zrow, zden, c1v,
                            as1.reshape(_N), ad1.reshape(_N), h1)

    h2, as2, ad2 = _mid(num1, den1[:, :, None], h1, as1, ad1, c1, mea,
                        bias1.reshape(1, _D), W2, att_src2.reshape(1, _D),
                        att_dst2.reshape(1, _D))

    num2, den2 = _edge_pass(src, dst, ea, zrow, zden, c2v,
                            as2.reshape(_N), ad2.reshape(_N), h2)

    gp, gl = _final(num2, den2[:, :, None], h2, as2, ad2, c2, mea,
                    bias2.reshape(1, _D), batch.reshape(_NB, 1, _BN),
                    S1_w, S1_b.reshape(1, _SOP), S2_w, S2_b.reshape(1, _SOP),
                    S3_w, S3_b.reshape(1, _SOP))
    return (gp, gl)
